# baseline (speedup 1.0000x reference)
import functools
import math

import jax
import jax.numpy as jnp
from jax import lax
from jax.experimental import pallas as pl
from jax.experimental.pallas import tpu as pltpu
from jax.experimental.pallas import tpu_sc as plsc

_EPS = 1e-5
_LANES = 16
_K = 56
_NBUF = 6
_G0 = 21
_G1 = 39


def _sc_aggregate(h, src, dst, pad_row):
    n, d_in = h.shape
    nw, ngroup, nbuf, k = src.shape
    info = plsc.get_sparse_core_info()
    nc, ns = info.num_cores, info.num_subcores
    assert nw == nc * ns and k % 8 == 0 and k <= 128 and nbuf == _NBUF
    assert ngroup == max(_G0, _G1) and _G0 % 3 == 0 and _G1 % 3 == 0
    rows_per_tile = -(-max(n, pad_row + 1) // (ns * 8)) * 8
    n_pad = rows_per_tile * ns
    assert pad_row < n_pad

    mesh = plsc.VectorSubcoreMesh(core_axis_name="c", subcore_axis_name="s")
    out_t = jax.ShapeDtypeStruct((n_pad, d_in), jnp.float32)

    @functools.partial(
        pl.kernel,
        mesh=mesh,
        out_type=[out_t, out_t],
        scratch_types=[
            [pltpu.VMEM((nbuf, k), jnp.int32) for _ in range(6)],
            [pltpu.VMEM((k, d_in), jnp.float32) for _ in range(nbuf)],
            pltpu.VMEM_SHARED((n_pad, d_in), jnp.float32),
            [pltpu.SemaphoreType.DMA for _ in range(3)],
            [pltpu.SemaphoreType.DMA for _ in range(nbuf)],
            pltpu.SemaphoreType.DMA,
        ],
    )
    def sc_kernel(h_hbm, src_hbm, dst_hbm, out0, out1,
                  idx_v, rows_v, acc_sh, sem_i, sem_g, sem_s):
        cid = lax.axis_index("c")
        sid = lax.axis_index("s")
        wid = sid * nc + cid
        src_a, dst_a, src_b, dst_b, src_c, dst_c = idx_v

        zvec = jnp.zeros((_LANES,), jnp.float32)
        segs = d_in // _LANES

        def zero_body(i, carry):
            rows_v[0][i // segs, pl.ds((i % segs) * _LANES, _LANES)] = zvec
            return carry

        lax.fori_loop(0, k * segs, zero_body, 0)
        row0 = sid * rows_per_tile
        r = 0
        while r < rows_per_tile:
            step = min(k, rows_per_tile - r)
            pltpu.sync_copy(rows_v[0].at[pl.ds(0, step)],
                            acc_sh.at[pl.ds(row0 + r, step)])
            r += step
        plsc.subcore_barrier()

        def fetch_idx(g, sbuf, dbuf, sem):
            return (pltpu.async_copy(src_hbm.at[wid, g], sbuf, sem),
                    pltpu.async_copy(dst_hbm.at[wid, g], dbuf, sem))

        def run_group(sbuf, dbuf, prev_sd):
            gd = []
            for b in range(nbuf):
                if prev_sd is not None:
                    prev_sd[b].wait()
                gd.append(pltpu.async_copy(
                    h_hbm.at[sbuf.at[b]], rows_v[b], sem_g[b]))
            sd = []
            for b in range(nbuf):
                gd[b].wait()
                sd.append(pltpu.async_copy(
                    rows_v[b], acc_sh.at[dbuf.at[b]], sem_s, add=True))
            return sd

        def tri_body(i, carry):
            g0 = i * 3
            fa = fetch_idx(g0, src_a, dst_a, sem_i[0])
            fa[0].wait()
            fa[1].wait()
            fb = fetch_idx(g0 + 1, src_b, dst_b, sem_i[1])
            sd_a = run_group(src_a, dst_a, None)
            fb[0].wait()
            fb[1].wait()
            fc = fetch_idx(g0 + 2, src_c, dst_c, sem_i[2])
            sd_b = run_group(src_b, dst_b, sd_a)
            fc[0].wait()
            fc[1].wait()
            sd_c = run_group(src_c, dst_c, sd_b)
            for b in range(nbuf):
                sd_c[b].wait()
            return carry

        @pl.when(cid == 0)
        def _():
            lax.fori_loop(0, _G0 // 3, tri_body, 0)

        @pl.when(cid == 1)
        def _():
            lax.fori_loop(0, _G1 // 3, tri_body, 0)

        plsc.subcore_barrier()

        rows = pl.ds(row0, rows_per_tile)

        @pl.when(cid == 0)
        def _():
            pltpu.sync_copy(acc_sh.at[rows], out0.at[rows])

        @pl.when(cid == 1)
        def _():
            pltpu.sync_copy(acc_sh.at[rows], out1.at[rows])

    return sc_kernel(h, src, dst)


def _tc_fuse(agg0, agg1, h, wfull, scale, shift):
    n, d_in = h.shape
    d_out = wfull.shape[1]
    br = 1000
    assert n % br == 0

    def body(a0_ref, a1_ref, h_ref, w_ref, sc_ref, sh_ref, o_ref):
        agg = a0_ref[...] + a1_ref[...]
        z = jnp.dot(agg, w_ref[...], preferred_element_type=jnp.float32,
                    precision=lax.Precision.HIGHEST)
        zb = z * sc_ref[...] + sh_ref[...]
        act = jnp.where(zb > 0.0, zb, jnp.exp(jnp.minimum(zb, 0.0)) - 1.0)
        o_ref[...] = h_ref[...] + act

    row_spec = pl.BlockSpec((br, d_in), lambda i: (i, 0))
    return pl.pallas_call(
        body,
        grid=(n // br,),
        in_specs=[
            row_spec,
            row_spec,
            row_spec,
            pl.BlockSpec((d_in, d_out), lambda i: (0, 0)),
            pl.BlockSpec((1, d_out), lambda i: (0, 0)),
            pl.BlockSpec((1, d_out), lambda i: (0, 0)),
        ],
        out_specs=pl.BlockSpec((br, d_out), lambda i: (i, 0)),
        out_shape=jax.ShapeDtypeStruct((n, d_out), jnp.float32),
    )(agg0, agg1, h, wfull, scale, shift)


def kernel(h, edge_index, e, W, gamma, beta):
    n, d_in = h.shape
    heads, d_out, _ = W.shape
    n_edges = edge_index.shape[1]

    nw, ns = 32, 16
    gsz = _K * _NBUF
    gmax = max(_G0, _G1)
    cap = ns * (_G0 + _G1) * gsz
    assert cap >= n_edges
    pad_row = n
    if cap > n_edges:
        pad = jnp.concatenate(
            [jnp.zeros((1, cap - n_edges), jnp.int32),
             jnp.full((1, cap - n_edges), pad_row, jnp.int32)])
        ei = jnp.concatenate([edge_index, pad], axis=1)
    else:
        ei = edge_index
    pad_grp = jnp.concatenate(
        [jnp.zeros((1, 1, gsz), jnp.int32),
         jnp.full((1, 1, gsz), pad_row, jnp.int32)])
    blocks, off = [], 0
    for w in range(nw):
        g_w = _G1 if (w % 2) else _G0
        cnt = g_w * gsz
        blk = ei[:, off:off + cnt].reshape(2, g_w, gsz)
        off += cnt
        if g_w < gmax:
            blk = jnp.concatenate(
                [blk, jnp.broadcast_to(pad_grp, (2, gmax - g_w, gsz))],
                axis=1)
        blocks.append(blk)
    eib = jnp.stack(blocks)
    ngroup = gmax
    src = eib[:, 0].reshape(nw, ngroup, _NBUF, _K)
    dst = eib[:, 1].reshape(nw, ngroup, _NBUF, _K)

    wfull = jnp.transpose(W, (2, 0, 1)).reshape(d_in, heads * d_out)
    inv_std = 1.0 / math.sqrt(1.0 + _EPS)
    scale = (gamma * inv_std).reshape(1, heads * d_out)
    shift = beta.reshape(1, heads * d_out)
    agg0, agg1 = _sc_aggregate(h, src, dst, pad_row)
    h_out = _tc_fuse(agg0, agg1, h, wfull, scale, shift)
    return (h_out, e)


# speedup vs baseline: 1.1250x; 1.1250x over previous
import functools
import math

import jax
import jax.numpy as jnp
from jax import lax
from jax.experimental import pallas as pl
from jax.experimental.pallas import tpu as pltpu
from jax.experimental.pallas import tpu_sc as plsc

_EPS = 1e-5
_LANES = 16
_K = 56
_NBUF = 6
_G0 = 39
_G1 = 21


def _sc_aggregate(h, src, dst, pad_row):
    n, d_in = h.shape
    nw, ngroup, nbuf, k = src.shape
    info = plsc.get_sparse_core_info()
    nc, ns = info.num_cores, info.num_subcores
    assert nw == nc * ns and k % 8 == 0 and k <= 128 and nbuf == _NBUF
    assert ngroup == max(_G0, _G1) and _G0 % 3 == 0 and _G1 % 3 == 0
    rows_per_tile = -(-max(n, pad_row + 1) // (ns * 8)) * 8
    n_pad = rows_per_tile * ns
    assert pad_row < n_pad

    mesh = plsc.VectorSubcoreMesh(core_axis_name="c", subcore_axis_name="s")
    out_t = jax.ShapeDtypeStruct((n_pad, d_in), jnp.float32)

    @functools.partial(
        pl.kernel,
        mesh=mesh,
        out_type=[out_t, out_t],
        scratch_types=[
            [pltpu.VMEM((nbuf, k), jnp.int32) for _ in range(6)],
            [pltpu.VMEM((k, d_in), jnp.float32) for _ in range(nbuf)],
            pltpu.VMEM_SHARED((n_pad, d_in), jnp.float32),
            [pltpu.SemaphoreType.DMA for _ in range(3)],
            [pltpu.SemaphoreType.DMA for _ in range(nbuf)],
            pltpu.SemaphoreType.DMA,
        ],
    )
    def sc_kernel(h_hbm, src_hbm, dst_hbm, out0, out1,
                  idx_v, rows_v, acc_sh, sem_i, sem_g, sem_s):
        cid = lax.axis_index("c")
        sid = lax.axis_index("s")
        wid = sid * nc + cid
        src_a, dst_a, src_b, dst_b, src_c, dst_c = idx_v

        zvec = jnp.zeros((_LANES,), jnp.float32)
        segs = d_in // _LANES

        def zero_body(i, carry):
            rows_v[0][i // segs, pl.ds((i % segs) * _LANES, _LANES)] = zvec
            return carry

        lax.fori_loop(0, k * segs, zero_body, 0)
        row0 = sid * rows_per_tile
        r = 0
        while r < rows_per_tile:
            step = min(k, rows_per_tile - r)
            pltpu.sync_copy(rows_v[0].at[pl.ds(0, step)],
                            acc_sh.at[pl.ds(row0 + r, step)])
            r += step
        plsc.subcore_barrier()

        def fetch_idx(g, sbuf, dbuf, sem):
            return (pltpu.async_copy(src_hbm.at[wid, g], sbuf, sem),
                    pltpu.async_copy(dst_hbm.at[wid, g], dbuf, sem))

        def run_group(sbuf, dbuf, prev_sd):
            gd = []
            for b in range(nbuf):
                if prev_sd is not None:
                    prev_sd[b].wait()
                gd.append(pltpu.async_copy(
                    h_hbm.at[sbuf.at[b]], rows_v[b], sem_g[b]))
            sd = []
            for b in range(nbuf):
                gd[b].wait()
                sd.append(pltpu.async_copy(
                    rows_v[b], acc_sh.at[dbuf.at[b]], sem_s, add=True))
            return sd

        def tri_body(i, carry):
            g0 = i * 3
            fa = fetch_idx(g0, src_a, dst_a, sem_i[0])
            fa[0].wait()
            fa[1].wait()
            fb = fetch_idx(g0 + 1, src_b, dst_b, sem_i[1])
            sd_a = run_group(src_a, dst_a, None)
            fb[0].wait()
            fb[1].wait()
            fc = fetch_idx(g0 + 2, src_c, dst_c, sem_i[2])
            sd_b = run_group(src_b, dst_b, sd_a)
            fc[0].wait()
            fc[1].wait()
            sd_c = run_group(src_c, dst_c, sd_b)
            for b in range(nbuf):
                sd_c[b].wait()
            return carry

        @pl.when(cid == 0)
        def _():
            lax.fori_loop(0, _G0 // 3, tri_body, 0)

        @pl.when(cid == 1)
        def _():
            lax.fori_loop(0, _G1 // 3, tri_body, 0)

        plsc.subcore_barrier()

        rows = pl.ds(row0, rows_per_tile)

        @pl.when(cid == 0)
        def _():
            pltpu.sync_copy(acc_sh.at[rows], out0.at[rows])

        @pl.when(cid == 1)
        def _():
            pltpu.sync_copy(acc_sh.at[rows], out1.at[rows])

    return sc_kernel(h, src, dst)


def _tc_fuse(agg0, agg1, h, wfull, scale, shift):
    n, d_in = h.shape
    d_out = wfull.shape[1]
    br = 1000
    assert n % br == 0

    def body(a0_ref, a1_ref, h_ref, w_ref, sc_ref, sh_ref, o_ref):
        agg = a0_ref[...] + a1_ref[...]
        z = jnp.dot(agg, w_ref[...], preferred_element_type=jnp.float32,
                    precision=lax.Precision.HIGHEST)
        zb = z * sc_ref[...] + sh_ref[...]
        act = jnp.where(zb > 0.0, zb, jnp.exp(jnp.minimum(zb, 0.0)) - 1.0)
        o_ref[...] = h_ref[...] + act

    row_spec = pl.BlockSpec((br, d_in), lambda i: (i, 0))
    return pl.pallas_call(
        body,
        grid=(n // br,),
        in_specs=[
            row_spec,
            row_spec,
            row_spec,
            pl.BlockSpec((d_in, d_out), lambda i: (0, 0)),
            pl.BlockSpec((1, d_out), lambda i: (0, 0)),
            pl.BlockSpec((1, d_out), lambda i: (0, 0)),
        ],
        out_specs=pl.BlockSpec((br, d_out), lambda i: (i, 0)),
        out_shape=jax.ShapeDtypeStruct((n, d_out), jnp.float32),
    )(agg0, agg1, h, wfull, scale, shift)


def kernel(h, edge_index, e, W, gamma, beta):
    n, d_in = h.shape
    heads, d_out, _ = W.shape
    n_edges = edge_index.shape[1]

    nw, ns = 32, 16
    gsz = _K * _NBUF
    gmax = max(_G0, _G1)
    cap = ns * (_G0 + _G1) * gsz
    assert cap >= n_edges
    pad_row = n
    if cap > n_edges:
        pad = jnp.concatenate(
            [jnp.zeros((1, cap - n_edges), jnp.int32),
             jnp.full((1, cap - n_edges), pad_row, jnp.int32)])
        ei = jnp.concatenate([edge_index, pad], axis=1)
    else:
        ei = edge_index
    pad_grp = jnp.concatenate(
        [jnp.zeros((1, 1, gsz), jnp.int32),
         jnp.full((1, 1, gsz), pad_row, jnp.int32)])
    blocks, off = [], 0
    for w in range(nw):
        g_w = _G1 if (w % 2) else _G0
        cnt = g_w * gsz
        blk = ei[:, off:off + cnt].reshape(2, g_w, gsz)
        off += cnt
        if g_w < gmax:
            blk = jnp.concatenate(
                [blk, jnp.broadcast_to(pad_grp, (2, gmax - g_w, gsz))],
                axis=1)
        blocks.append(blk)
    eib = jnp.stack(blocks)
    ngroup = gmax
    src = eib[:, 0].reshape(nw, ngroup, _NBUF, _K)
    dst = eib[:, 1].reshape(nw, ngroup, _NBUF, _K)

    wfull = jnp.transpose(W, (2, 0, 1)).reshape(d_in, heads * d_out)
    inv_std = 1.0 / math.sqrt(1.0 + _EPS)
    scale = (gamma * inv_std).reshape(1, heads * d_out)
    shift = beta.reshape(1, heads * d_out)
    agg0, agg1 = _sc_aggregate(h, src, dst, pad_row)
    h_out = _tc_fuse(agg0, agg1, h, wfull, scale, shift)
    return (h_out, e)


# speedup vs baseline: 1.1616x; 1.0325x over previous
import functools
import math

import jax
import jax.numpy as jnp
from jax import lax
from jax.experimental import pallas as pl
from jax.experimental.pallas import tpu as pltpu
from jax.experimental.pallas import tpu_sc as plsc

_EPS = 1e-5
_LANES = 16
_K = 56
_NBUF = 6
_G0 = 42
_G1 = 18


def _sc_aggregate(h, src, dst, pad_row):
    n, d_in = h.shape
    nw, ngroup, nbuf, k = src.shape
    info = plsc.get_sparse_core_info()
    nc, ns = info.num_cores, info.num_subcores
    assert nw == nc * ns and k % 8 == 0 and k <= 128 and nbuf == _NBUF
    assert ngroup == max(_G0, _G1) and _G0 % 3 == 0 and _G1 % 3 == 0
    rows_per_tile = -(-max(n, pad_row + 1) // (ns * 8)) * 8
    n_pad = rows_per_tile * ns
    assert pad_row < n_pad

    mesh = plsc.VectorSubcoreMesh(core_axis_name="c", subcore_axis_name="s")
    out_t = jax.ShapeDtypeStruct((n_pad, d_in), jnp.float32)

    @functools.partial(
        pl.kernel,
        mesh=mesh,
        out_type=[out_t, out_t],
        scratch_types=[
            [pltpu.VMEM((nbuf, k), jnp.int32) for _ in range(6)],
            [pltpu.VMEM((k, d_in), jnp.float32) for _ in range(nbuf)],
            pltpu.VMEM_SHARED((n_pad, d_in), jnp.float32),
            [pltpu.SemaphoreType.DMA for _ in range(3)],
            [pltpu.SemaphoreType.DMA for _ in range(nbuf)],
            pltpu.SemaphoreType.DMA,
        ],
    )
    def sc_kernel(h_hbm, src_hbm, dst_hbm, out0, out1,
                  idx_v, rows_v, acc_sh, sem_i, sem_g, sem_s):
        cid = lax.axis_index("c")
        sid = lax.axis_index("s")
        wid = sid * nc + cid
        src_a, dst_a, src_b, dst_b, src_c, dst_c = idx_v

        zvec = jnp.zeros((_LANES,), jnp.float32)
        segs = d_in // _LANES

        def zero_body(i, carry):
            rows_v[0][i // segs, pl.ds((i % segs) * _LANES, _LANES)] = zvec
            return carry

        lax.fori_loop(0, k * segs, zero_body, 0)
        row0 = sid * rows_per_tile
        r = 0
        while r < rows_per_tile:
            step = min(k, rows_per_tile - r)
            pltpu.sync_copy(rows_v[0].at[pl.ds(0, step)],
                            acc_sh.at[pl.ds(row0 + r, step)])
            r += step
        plsc.subcore_barrier()

        def fetch_idx(g, sbuf, dbuf, sem):
            return (pltpu.async_copy(src_hbm.at[wid, g], sbuf, sem),
                    pltpu.async_copy(dst_hbm.at[wid, g], dbuf, sem))

        def run_group(sbuf, dbuf, prev_sd):
            gd = []
            for b in range(nbuf):
                if prev_sd is not None:
                    prev_sd[b].wait()
                gd.append(pltpu.async_copy(
                    h_hbm.at[sbuf.at[b]], rows_v[b], sem_g[b]))
            sd = []
            for b in range(nbuf):
                gd[b].wait()
                sd.append(pltpu.async_copy(
                    rows_v[b], acc_sh.at[dbuf.at[b]], sem_s, add=True))
            return sd

        def tri_body(i, carry):
            g0 = i * 3
            fa = fetch_idx(g0, src_a, dst_a, sem_i[0])
            fa[0].wait()
            fa[1].wait()
            fb = fetch_idx(g0 + 1, src_b, dst_b, sem_i[1])
            sd_a = run_group(src_a, dst_a, None)
            fb[0].wait()
            fb[1].wait()
            fc = fetch_idx(g0 + 2, src_c, dst_c, sem_i[2])
            sd_b = run_group(src_b, dst_b, sd_a)
            fc[0].wait()
            fc[1].wait()
            sd_c = run_group(src_c, dst_c, sd_b)
            for b in range(nbuf):
                sd_c[b].wait()
            return carry

        @pl.when(cid == 0)
        def _():
            lax.fori_loop(0, _G0 // 3, tri_body, 0)

        @pl.when(cid == 1)
        def _():
            lax.fori_loop(0, _G1 // 3, tri_body, 0)

        plsc.subcore_barrier()

        rows = pl.ds(row0, rows_per_tile)

        @pl.when(cid == 0)
        def _():
            pltpu.sync_copy(acc_sh.at[rows], out0.at[rows])

        @pl.when(cid == 1)
        def _():
            pltpu.sync_copy(acc_sh.at[rows], out1.at[rows])

    return sc_kernel(h, src, dst)


def _tc_fuse(agg0, agg1, h, wfull, scale, shift):
    n, d_in = h.shape
    d_out = wfull.shape[1]
    br = 1000
    assert n % br == 0

    def body(a0_ref, a1_ref, h_ref, w_ref, sc_ref, sh_ref, o_ref):
        agg = a0_ref[...] + a1_ref[...]
        z = jnp.dot(agg, w_ref[...], preferred_element_type=jnp.float32,
                    precision=lax.Precision.HIGHEST)
        zb = z * sc_ref[...] + sh_ref[...]
        act = jnp.where(zb > 0.0, zb, jnp.exp(jnp.minimum(zb, 0.0)) - 1.0)
        o_ref[...] = h_ref[...] + act

    row_spec = pl.BlockSpec((br, d_in), lambda i: (i, 0))
    return pl.pallas_call(
        body,
        grid=(n // br,),
        in_specs=[
            row_spec,
            row_spec,
            row_spec,
            pl.BlockSpec((d_in, d_out), lambda i: (0, 0)),
            pl.BlockSpec((1, d_out), lambda i: (0, 0)),
            pl.BlockSpec((1, d_out), lambda i: (0, 0)),
        ],
        out_specs=pl.BlockSpec((br, d_out), lambda i: (i, 0)),
        out_shape=jax.ShapeDtypeStruct((n, d_out), jnp.float32),
    )(agg0, agg1, h, wfull, scale, shift)


def kernel(h, edge_index, e, W, gamma, beta):
    n, d_in = h.shape
    heads, d_out, _ = W.shape
    n_edges = edge_index.shape[1]

    nw, ns = 32, 16
    gsz = _K * _NBUF
    gmax = max(_G0, _G1)
    cap = ns * (_G0 + _G1) * gsz
    assert cap >= n_edges
    pad_row = n
    if cap > n_edges:
        pad = jnp.concatenate(
            [jnp.zeros((1, cap - n_edges), jnp.int32),
             jnp.full((1, cap - n_edges), pad_row, jnp.int32)])
        ei = jnp.concatenate([edge_index, pad], axis=1)
    else:
        ei = edge_index
    pad_grp = jnp.concatenate(
        [jnp.zeros((1, 1, gsz), jnp.int32),
         jnp.full((1, 1, gsz), pad_row, jnp.int32)])
    blocks, off = [], 0
    for w in range(nw):
        g_w = _G1 if (w % 2) else _G0
        cnt = g_w * gsz
        blk = ei[:, off:off + cnt].reshape(2, g_w, gsz)
        off += cnt
        if g_w < gmax:
            blk = jnp.concatenate(
                [blk, jnp.broadcast_to(pad_grp, (2, gmax - g_w, gsz))],
                axis=1)
        blocks.append(blk)
    eib = jnp.stack(blocks)
    ngroup = gmax
    src = eib[:, 0].reshape(nw, ngroup, _NBUF, _K)
    dst = eib[:, 1].reshape(nw, ngroup, _NBUF, _K)

    wfull = jnp.transpose(W, (2, 0, 1)).reshape(d_in, heads * d_out)
    inv_std = 1.0 / math.sqrt(1.0 + _EPS)
    scale = (gamma * inv_std).reshape(1, heads * d_out)
    shift = beta.reshape(1, heads * d_out)
    agg0, agg1 = _sc_aggregate(h, src, dst, pad_row)
    h_out = _tc_fuse(agg0, agg1, h, wfull, scale, shift)
    return (h_out, e)
